# Initial kernel scaffold; baseline (speedup 1.0000x reference)
import jax
import jax.numpy as jnp
from jax.experimental import pallas as pl


def kernel(x):
    raise NotImplementedError("write your pallas kernel here")


# speedup vs baseline: 4.4198x; 4.4198x over previous
import functools

import numpy as np
import jax
import jax.numpy as jnp
from jax.experimental import pallas as pl
from jax.experimental.pallas import tpu as pltpu

_SCALE = 2


def _keys_cubic(t):
    t = np.abs(np.asarray(t, np.float64))
    return np.where(
        t <= 1.0,
        (1.5 * t - 2.5) * t * t + 1.0,
        np.where(t < 2.0, ((-0.5 * t + 2.5) * t - 4.0) * t + 2.0, 0.0),
    )


def _tap_weights(scale):
    offs = [(scale - 1) / (2.0 * scale) - j / float(scale) for j in range(scale)]
    wk = np.stack([_keys_cubic([b - 2, b - 1, b, b + 1, b + 2]) for b in offs])
    wk = wk / wk.sum(axis=1, keepdims=True)
    return wk.astype(np.float32)


def _h_matrix(w, scale):
    wk = _tap_weights(scale)
    m = np.zeros((w, w * scale), np.float32)
    cols = np.arange(w)
    for q in range(5):
        src = np.clip(cols + q - 2, 0, w - 1)
        for j in range(scale):
            np.add.at(m, (src, cols * scale + j), wk[j, q])
    return m


def _upsample2x_kernel(x_ref, mh_ref, o_ref, *, wk):
    tile, h, w = x_ref.shape
    x = x_ref[...]

    first = x[:, :1, :]
    last = x[:, h - 1 :, :]
    shifts = (
        jnp.concatenate([first, first, x[:, : h - 2, :]], axis=1),
        jnp.concatenate([first, x[:, : h - 1, :]], axis=1),
        x,
        jnp.concatenate([x[:, 1:, :], last], axis=1),
        jnp.concatenate([x[:, 2:, :], last, last], axis=1),
    )
    u0 = shifts[0] * wk[0][0]
    u1 = shifts[0] * wk[1][0]
    for q in range(1, 5):
        u0 = u0 + shifts[q] * wk[0][q]
        u1 = u1 + shifts[q] * wk[1][q]

    v = jnp.stack([u0, u1], axis=2).reshape(tile, 2 * h, w)
    vb = v.astype(jnp.bfloat16).reshape(tile * 2 * h, w)
    y = jnp.dot(vb, mh_ref[...], preferred_element_type=jnp.float32)
    o_ref[...] = y.reshape(tile, 2 * h, 2 * w)


def kernel(x):
    b, c, h, w = x.shape
    scale = _SCALE
    hs, ws = h * scale, w * scale
    bc = b * c

    xr = x.reshape(bc, h, w)
    mh = jnp.asarray(_h_matrix(w, scale).astype(np.float32)).astype(jnp.bfloat16)
    wk = tuple(tuple(float(v) for v in row) for row in _tap_weights(scale))

    tile = 64
    while bc % tile:
        tile //= 2
    grid = (bc // tile,)

    out = pl.pallas_call(
        functools.partial(_upsample2x_kernel, wk=wk),
        out_shape=jax.ShapeDtypeStruct((bc, hs, ws), x.dtype),
        grid=grid,
        in_specs=[
            pl.BlockSpec((tile, h, w), lambda i: (i, 0, 0)),
            pl.BlockSpec((w, ws), lambda i: (0, 0)),
        ],
        out_specs=pl.BlockSpec((tile, hs, ws), lambda i: (i, 0, 0)),
        compiler_params=pltpu.CompilerParams(
            dimension_semantics=("parallel",),
            vmem_limit_bytes=100 * 1024 * 1024,
        ),
    )(xr, mh)

    return out.reshape(b, c, hs, ws)
